# baseline (speedup 1.0000x reference)
import functools
import jax
import jax.numpy as jnp
from jax import lax
from jax.experimental import pallas as pl
from jax.experimental.pallas import tpu as pltpu
from jax.experimental.pallas import tpu_sc as plsc

VOCAB = 100000
EMB = 64
REGION = 5
BATCH = 1024
SEQ = 200

NC = 2
NS = 16
NW = NC * NS
ROWS_PER_W = BATCH // NW
LANES = 16
GROUPS = EMB // LANES
UROW = 384
CH = 40
NCH = SEQ // CH
WOFF = 8
WROWS = 216


RNB = 40
RCHUNKS = VOCAB // RNB
RSTEPS = 80


def _relayout_body(u_hbm, u2_hbm, in_v, out_v, sem_i0, sem_i1, sem_o0, sem_o1):
    c = lax.axis_index("c")
    s = lax.axis_index("s")
    wid = s * NC + c
    sem_is = (sem_i0, sem_i1)
    sem_os = (sem_o0, sem_o1)

    def start_in(k, buf):
        cid = wid + NW * k

        @pl.when(cid < RCHUNKS)
        def _():
            pltpu.async_copy(
                u_hbm.at[pl.ds(cid * RNB, RNB)], in_v.at[buf], sem_is[buf])

    def do_chunk(k, buf):
        cid = wid + NW * k

        @pl.when(cid < RCHUNKS)
        def _():
            pltpu.make_async_copy(
                u_hbm.at[pl.ds(0, RNB)], in_v.at[buf], sem_is[buf]).wait()

            @pl.when(k >= 2)
            def _():
                pltpu.make_async_copy(
                    out_v.at[buf], u2_hbm.at[pl.ds(0, RNB)], sem_os[buf]).wait()

            def row(t, carry):
                for g in range(REGION * GROUPS):
                    out_v[buf, t, pl.ds(g * LANES, LANES)] = (
                        in_v[buf, t, pl.ds(g * LANES, LANES)])
                return carry

            lax.fori_loop(0, RNB, row, 0, unroll=2)

            pltpu.async_copy(
                out_v.at[buf], u2_hbm.at[pl.ds(cid * RNB, RNB)], sem_os[buf])

    start_in(0, 0)
    start_in(1, 1)

    def step(j, carry):
        k0 = 2 * j
        do_chunk(k0, 0)
        start_in(k0 + 2, 0)
        do_chunk(k0 + 1, 1)
        start_in(k0 + 3, 1)
        return carry

    lax.fori_loop(0, RSTEPS // 2, step, 0)

    for buf in range(2):
        pltpu.make_async_copy(
            out_v.at[buf], u2_hbm.at[pl.ds(0, RNB)], sem_os[buf]).wait()


def _sc_body(seq_hbm, semb_hbm, u_hbm, out_hbm, idx0_v, idx1_v, rows_v, win_v,
             out_v, sem_seq, sem_win, sem_g0, sem_g1, sem_o0, sem_o1):
    cc = lax.axis_index("c")
    s = lax.axis_index("s")
    wid = s * NC + cc
    row0 = wid * ROWS_PER_W

    zeros = jnp.zeros((LANES,), jnp.float32)
    for q in range(2):
        for prow in (WOFF - 2, WOFF - 1, WOFF + SEQ, WOFF + SEQ + 1):
            for g in range(GROUPS):
                win_v[q, prow, pl.ds(g * LANES, LANES)] = zeros

    idxbufs = (idx0_v, idx1_v)
    sem_gs = (sem_g0, sem_g1)
    sem_os = (sem_o0, sem_o1)

    def start_gather(qidx, ci, p):
        pltpu.async_copy(
            u_hbm.at[idxbufs[qidx].at[pl.ds(ci * CH, CH)]], rows_v.at[p],
            sem_gs[p])

    def wait_gather(p):
        pltpu.make_async_copy(
            u_hbm.at[pl.ds(0, CH)], rows_v.at[p], sem_gs[p]).wait()

    def wait_out(p):
        pltpu.make_async_copy(
            out_v.at[p], out_hbm.at[0, pl.ds(0, CH)], sem_os[p]).wait()

    def compute_chunk(q, ci, p, row):
        woff0 = ci * CH + WOFF - 2
        init = []
        for g in range(GROUPS):
            slg = pl.ds(g * LANES, LANES)
            for d in range(4):
                init.append(win_v[q, woff0 + d, slg])

        def tok(t, carry):
            new = []
            for g in range(GROUPS):
                slg = pl.ds(g * LANES, LANES)
                wa, wb, wc, wd = carry[4 * g:4 * g + 4]
                we = win_v[q, woff0 + 4 + t, slg]
                ws = (wa, wb, wc, wd, we)
                acc = wa * rows_v[p, t, pl.ds(g * LANES, LANES)]
                for r in range(1, REGION):
                    u = rows_v[p, t, pl.ds(r * EMB + g * LANES, LANES)]
                    acc = jnp.maximum(acc, ws[r] * u)
                out_v[p, t, slg] = acc
                new.extend([wb, wc, wd, we])
            return tuple(new)

        lax.fori_loop(0, CH, tok, tuple(init), unroll=4)
        pltpu.async_copy(
            out_v.at[p], out_hbm.at[row, pl.ds(ci * CH, CH)], sem_os[p])

    pltpu.sync_copy(seq_hbm.at[row0], idx0_v)
    pltpu.async_copy(semb_hbm.at[row0], win_v.at[0, pl.ds(WOFF, SEQ)], sem_win)
    start_gather(0, 0, 0)

    def step(j, carry):
        for c in range(2):
            row = row0 + 2 * j + c
            nxt = 2 * j + c + 1

            pltpu.make_async_copy(
                semb_hbm.at[row0], win_v.at[c, pl.ds(WOFF, SEQ)], sem_win).wait()

            def prefetch_next():
                pltpu.async_copy(seq_hbm.at[row + 1], idxbufs[1 - c], sem_seq)
                pltpu.async_copy(
                    semb_hbm.at[row + 1], win_v.at[1 - c, pl.ds(WOFF, SEQ)],
                    sem_win)

            if c == 0:
                prefetch_next()
            else:
                @pl.when(j < ROWS_PER_W // 2 - 1)
                def _():
                    prefetch_next()

            for ci in range(NCH):
                p = (c + ci) % 2

                if ci < NCH - 1:
                    start_gather(c, ci + 1, 1 - p)
                else:
                    def next_row_gather():
                        pltpu.make_async_copy(
                            seq_hbm.at[row0], idxbufs[1 - c], sem_seq).wait()
                        start_gather(1 - c, 0, 1 - p)

                    if c == 0:
                        next_row_gather()
                    else:
                        @pl.when(j < ROWS_PER_W // 2 - 1)
                        def _():
                            next_row_gather()

                wait_gather(p)

                if c == 0 and ci < 2:
                    @pl.when(j > 0)
                    def _():
                        wait_out(p)
                else:
                    wait_out(p)

                compute_chunk(c, ci, p, row)
        return carry

    lax.fori_loop(0, ROWS_PER_W // 2, step, 0)
    wait_out(0)
    wait_out(1)


@jax.jit
def _region_embed(seq, seq_emb, U):
    seq2 = seq.astype(jnp.int32)
    mesh = plsc.VectorSubcoreMesh(core_axis_name="c", subcore_axis_name="s")
    relayout = pl.kernel(
        _relayout_body,
        out_type=jax.ShapeDtypeStruct((VOCAB, UROW), jnp.float32),
        mesh=mesh,
        scratch_types=[
            pltpu.VMEM((2, RNB, REGION * EMB), jnp.float32),
            pltpu.VMEM((2, RNB, UROW), jnp.float32),
            pltpu.SemaphoreType.DMA,
            pltpu.SemaphoreType.DMA,
            pltpu.SemaphoreType.DMA,
            pltpu.SemaphoreType.DMA,
        ],
    )
    u2 = relayout(U.reshape(VOCAB, REGION * EMB))
    f = pl.kernel(
        _sc_body,
        out_type=jax.ShapeDtypeStruct((BATCH, SEQ, EMB), jnp.float32),
        mesh=mesh,
        scratch_types=[
            pltpu.VMEM((SEQ,), jnp.int32),
            pltpu.VMEM((SEQ,), jnp.int32),
            pltpu.VMEM((2, CH, UROW), jnp.float32),
            pltpu.VMEM((2, WROWS, EMB), jnp.float32),
            pltpu.VMEM((2, CH, EMB), jnp.float32),
            pltpu.SemaphoreType.DMA,
            pltpu.SemaphoreType.DMA,
            pltpu.SemaphoreType.DMA,
            pltpu.SemaphoreType.DMA,
            pltpu.SemaphoreType.DMA,
            pltpu.SemaphoreType.DMA,
        ],
    )
    return f(seq2, seq_emb, u2)


def kernel(seq, seq_emb, U):
    return _region_embed(seq, seq_emb, U)


# speedup vs baseline: 1.0270x; 1.0270x over previous
import functools
import jax
import jax.numpy as jnp
from jax import lax
from jax.experimental import pallas as pl
from jax.experimental.pallas import tpu as pltpu
from jax.experimental.pallas import tpu_sc as plsc

VOCAB = 100000
EMB = 64
REGION = 5
BATCH = 1024
SEQ = 200

NC = 2
NS = 16
NW = NC * NS
ROWS_PER_W = BATCH // NW
LANES = 16
GROUPS = EMB // LANES
UROW = 384
CH = 40
NCH = SEQ // CH
WOFF = 8
WROWS = 216


RNB = 40
RCHUNKS = VOCAB // RNB
RSTEPS = 80


def _relayout_body(u_hbm, u2_hbm, in_v, out_v, sem_i0, sem_i1, sem_o0, sem_o1):
    c = lax.axis_index("c")
    s = lax.axis_index("s")
    wid = s * NC + c
    sem_is = (sem_i0, sem_i1)
    sem_os = (sem_o0, sem_o1)

    def start_in(k, buf):
        cid = wid + NW * k

        @pl.when(cid < RCHUNKS)
        def _():
            pltpu.async_copy(
                u_hbm.at[pl.ds(cid * RNB, RNB)], in_v.at[buf], sem_is[buf])

    def do_chunk(k, buf):
        cid = wid + NW * k

        @pl.when(cid < RCHUNKS)
        def _():
            pltpu.make_async_copy(
                u_hbm.at[pl.ds(0, RNB)], in_v.at[buf], sem_is[buf]).wait()

            @pl.when(k >= 2)
            def _():
                pltpu.make_async_copy(
                    out_v.at[buf], u2_hbm.at[pl.ds(0, RNB)], sem_os[buf]).wait()

            def row(t, carry):
                for g in range(REGION * GROUPS):
                    out_v[buf, t, pl.ds(g * LANES, LANES)] = (
                        in_v[buf, t, pl.ds(g * LANES, LANES)])
                return carry

            lax.fori_loop(0, RNB, row, 0, unroll=2)

            pltpu.async_copy(
                out_v.at[buf], u2_hbm.at[pl.ds(cid * RNB, RNB)], sem_os[buf])

    start_in(0, 0)
    start_in(1, 1)

    def step(j, carry):
        k0 = 2 * j
        do_chunk(k0, 0)
        start_in(k0 + 2, 0)
        do_chunk(k0 + 1, 1)
        start_in(k0 + 3, 1)
        return carry

    lax.fori_loop(0, RSTEPS // 2, step, 0)

    for buf in range(2):
        pltpu.make_async_copy(
            out_v.at[buf], u2_hbm.at[pl.ds(0, RNB)], sem_os[buf]).wait()


def _sc_body(seq_hbm, semb_hbm, u_hbm, out_hbm, idx0_v, idx1_v, rows_v, win_v,
             out_v, sem_seq, sem_win, sem_g0, sem_g1, sem_o0, sem_o1):
    cc = lax.axis_index("c")
    s = lax.axis_index("s")
    wid = s * NC + cc
    row0 = wid * ROWS_PER_W

    zeros = jnp.zeros((LANES,), jnp.float32)
    for q in range(2):
        for prow in (WOFF - 2, WOFF - 1, WOFF + SEQ, WOFF + SEQ + 1):
            for g in range(GROUPS):
                win_v[q, prow, pl.ds(g * LANES, LANES)] = zeros

    idxbufs = (idx0_v, idx1_v)
    sem_gs = (sem_g0, sem_g1)
    sem_os = (sem_o0, sem_o1)

    def start_gather(qidx, ci, p):
        pltpu.async_copy(
            u_hbm.at[idxbufs[qidx].at[pl.ds(ci * CH, CH)]], rows_v.at[p],
            sem_gs[p])

    def wait_gather(p):
        pltpu.make_async_copy(
            u_hbm.at[pl.ds(0, CH)], rows_v.at[p], sem_gs[p]).wait()

    def wait_out(p):
        pltpu.make_async_copy(
            out_v.at[p], out_hbm.at[0, pl.ds(0, CH)], sem_os[p]).wait()

    def compute_chunk(q, ci, p, row):
        woff0 = ci * CH + WOFF - 2
        init = []
        for g in range(GROUPS):
            slg = pl.ds(g * LANES, LANES)
            for d in range(4):
                init.append(win_v[q, woff0 + d, slg])

        def tok(t, carry):
            new = []
            for g in range(GROUPS):
                slg = pl.ds(g * LANES, LANES)
                wa, wb, wc, wd = carry[4 * g:4 * g + 4]
                we = win_v[q, woff0 + 4 + t, slg]
                ws = (wa, wb, wc, wd, we)
                acc = wa * rows_v[p, t, pl.ds(g * LANES, LANES)]
                for r in range(1, REGION):
                    u = rows_v[p, t, pl.ds(r * EMB + g * LANES, LANES)]
                    acc = jnp.maximum(acc, ws[r] * u)
                out_v[p, t, slg] = acc
                new.extend([wb, wc, wd, we])
            return tuple(new)

        lax.fori_loop(0, CH, tok, tuple(init), unroll=2)
        pltpu.async_copy(
            out_v.at[p], out_hbm.at[row, pl.ds(ci * CH, CH)], sem_os[p])

    pltpu.sync_copy(seq_hbm.at[row0], idx0_v)
    pltpu.async_copy(semb_hbm.at[row0], win_v.at[0, pl.ds(WOFF, SEQ)], sem_win)
    start_gather(0, 0, 0)

    def step(j, carry):
        for c in range(2):
            row = row0 + 2 * j + c
            nxt = 2 * j + c + 1

            pltpu.make_async_copy(
                semb_hbm.at[row0], win_v.at[c, pl.ds(WOFF, SEQ)], sem_win).wait()

            def prefetch_next():
                pltpu.async_copy(seq_hbm.at[row + 1], idxbufs[1 - c], sem_seq)
                pltpu.async_copy(
                    semb_hbm.at[row + 1], win_v.at[1 - c, pl.ds(WOFF, SEQ)],
                    sem_win)

            if c == 0:
                prefetch_next()
            else:
                @pl.when(j < ROWS_PER_W // 2 - 1)
                def _():
                    prefetch_next()

            for ci in range(NCH):
                p = (c + ci) % 2

                if ci < NCH - 1:
                    start_gather(c, ci + 1, 1 - p)
                else:
                    def next_row_gather():
                        pltpu.make_async_copy(
                            seq_hbm.at[row0], idxbufs[1 - c], sem_seq).wait()
                        start_gather(1 - c, 0, 1 - p)

                    if c == 0:
                        next_row_gather()
                    else:
                        @pl.when(j < ROWS_PER_W // 2 - 1)
                        def _():
                            next_row_gather()

                wait_gather(p)

                if c == 0 and ci < 2:
                    @pl.when(j > 0)
                    def _():
                        wait_out(p)
                else:
                    wait_out(p)

                compute_chunk(c, ci, p, row)
        return carry

    lax.fori_loop(0, ROWS_PER_W // 2, step, 0)
    wait_out(0)
    wait_out(1)


@jax.jit
def _region_embed(seq, seq_emb, U):
    seq2 = seq.astype(jnp.int32)
    mesh = plsc.VectorSubcoreMesh(core_axis_name="c", subcore_axis_name="s")
    relayout = pl.kernel(
        _relayout_body,
        out_type=jax.ShapeDtypeStruct((VOCAB, UROW), jnp.float32),
        mesh=mesh,
        scratch_types=[
            pltpu.VMEM((2, RNB, REGION * EMB), jnp.float32),
            pltpu.VMEM((2, RNB, UROW), jnp.float32),
            pltpu.SemaphoreType.DMA,
            pltpu.SemaphoreType.DMA,
            pltpu.SemaphoreType.DMA,
            pltpu.SemaphoreType.DMA,
        ],
    )
    u2 = relayout(U.reshape(VOCAB, REGION * EMB))
    f = pl.kernel(
        _sc_body,
        out_type=jax.ShapeDtypeStruct((BATCH, SEQ, EMB), jnp.float32),
        mesh=mesh,
        scratch_types=[
            pltpu.VMEM((SEQ,), jnp.int32),
            pltpu.VMEM((SEQ,), jnp.int32),
            pltpu.VMEM((2, CH, UROW), jnp.float32),
            pltpu.VMEM((2, WROWS, EMB), jnp.float32),
            pltpu.VMEM((2, CH, EMB), jnp.float32),
            pltpu.SemaphoreType.DMA,
            pltpu.SemaphoreType.DMA,
            pltpu.SemaphoreType.DMA,
            pltpu.SemaphoreType.DMA,
            pltpu.SemaphoreType.DMA,
            pltpu.SemaphoreType.DMA,
        ],
    )
    return f(seq2, seq_emb, u2)


def kernel(seq, seq_emb, U):
    return _region_embed(seq, seq_emb, U)
